# Initial kernel scaffold; baseline (speedup 1.0000x reference)
import functools

import jax
import jax.numpy as jnp
from jax import lax
from jax.experimental import pallas as pl
from jax.experimental.pallas import tpu as pltpu

_EPS = 1e-5
_VMEM = 100 * 1024 * 1024


def _ceil_to(x, m):
    return (x + m - 1) // m * m


def _gemm_stats_body(p_ref, w_ref, b_ref, y_ref, s1_ref, s2_ref):
    m = pl.program_id(1)
    val = jnp.dot(p_ref[...], w_ref[...],
                  preferred_element_type=jnp.float32) + b_ref[...]

    @pl.when(m == 0)
    def _():
        s1_ref[...] = jnp.zeros_like(s1_ref)
        s2_ref[...] = jnp.zeros_like(s2_ref)

    s1_ref[0] = s1_ref[0] + jnp.sum(val, axis=0, keepdims=True)
    s2_ref[0] = s2_ref[0] + jnp.sum(val * val, axis=0, keepdims=True)
    y_ref[...] = val


def _gemm_act_body(act, p_ref, w_ref, b_ref, y_ref):
    val = jnp.dot(p_ref[...], w_ref[...],
                  preferred_element_type=jnp.float32) + b_ref[...]
    if act == 'relu':
        val = jnp.maximum(val, 0.0)
    elif act == 'tanh':
        val = jnp.tanh(val)
    y_ref[...] = val


def _norm_body(act, add_res, inv_m, y_ref, s1_ref, s2_ref, *rest):
    if add_res:
        r_ref, o_ref = rest
    else:
        (o_ref,) = rest
    mean = (s1_ref[0, 0] + s1_ref[1, 0]) * inv_m
    ex2 = (s2_ref[0, 0] + s2_ref[1, 0]) * inv_m
    var = jnp.maximum(ex2 - mean * mean, 0.0)
    val = (y_ref[...] - mean) * lax.rsqrt(var + _EPS)
    if add_res:
        val = val + r_ref[...]
    if act == 'relu':
        val = jnp.maximum(val, 0.0)
    elif act == 'tanh':
        val = jnp.tanh(val)
    o_ref[...] = val


def _pick_tm(m_half):
    for cand in (2048, 1024, 512):
        if m_half % cand == 0:
            return cand
    return m_half


def _gemm(patches, w_mat, bias, *, do_norm, act, res, m_true):
    M, Kp = patches.shape
    Cp = w_mat.shape[1]
    tm = _pick_tm(M // 2)
    n_mt = M // (2 * tm)
    b2 = bias.reshape(1, Cp)

    p_spec = pl.BlockSpec((tm, Kp), lambda c, m: (c * n_mt + m, 0))
    w_spec = pl.BlockSpec((Kp, Cp), lambda c, m: (0, 0))
    b_spec = pl.BlockSpec((1, Cp), lambda c, m: (0, 0))
    y_spec = pl.BlockSpec((tm, Cp), lambda c, m: (c * n_mt + m, 0))
    params = pltpu.CompilerParams(
        dimension_semantics=("core_parallel", "arbitrary"),
        vmem_limit_bytes=_VMEM)

    if not do_norm:
        return pl.pallas_call(
            functools.partial(_gemm_act_body, act),
            grid=(2, n_mt),
            in_specs=[p_spec, w_spec, b_spec],
            out_specs=y_spec,
            out_shape=jax.ShapeDtypeStruct((M, Cp), jnp.float32),
            compiler_params=params,
        )(patches, w_mat, b2)

    stat_spec = pl.BlockSpec((1, 1, Cp), lambda c, m: (c, 0, 0))
    y, s1, s2 = pl.pallas_call(
        _gemm_stats_body,
        grid=(2, n_mt),
        in_specs=[p_spec, w_spec, b_spec],
        out_specs=(y_spec, stat_spec, stat_spec),
        out_shape=(
            jax.ShapeDtypeStruct((M, Cp), jnp.float32),
            jax.ShapeDtypeStruct((2, 1, Cp), jnp.float32),
            jax.ShapeDtypeStruct((2, 1, Cp), jnp.float32),
        ),
        compiler_params=params,
    )(patches, w_mat, b2)

    norm_specs = [
        y_spec,
        pl.BlockSpec((2, 1, Cp), lambda c, m: (0, 0, 0)),
        pl.BlockSpec((2, 1, Cp), lambda c, m: (0, 0, 0)),
    ]
    args = [y, s1, s2]
    if res is not None:
        norm_specs.append(y_spec)
        args.append(res)
    return pl.pallas_call(
        functools.partial(_norm_body, act, res is not None, 1.0 / m_true),
        grid=(2, n_mt),
        in_specs=norm_specs,
        out_specs=y_spec,
        out_shape=jax.ShapeDtypeStruct((M, Cp), jnp.float32),
        compiler_params=pltpu.CompilerParams(
            dimension_semantics=("core_parallel", "arbitrary"),
            vmem_limit_bytes=_VMEM),
    )(*args)


def _pad_hw(x, p, mode):
    if p == 0:
        return x
    w = ((p, p), (p, p), (0, 0))
    return jnp.pad(x, w, mode='reflect' if mode == 'reflect' else 'constant')


def _im2col(xp, k, stride):
    H, W, C = xp.shape
    Ho = (H - k) // stride + 1
    Wo = (W - k) // stride + 1
    cols = [xp[i:i + Ho * stride:stride, j:j + Wo * stride:stride, :]
            for i in range(k) for j in range(k)]
    patches = jnp.stack(cols, axis=2).reshape(Ho * Wo, k * k * C)
    return patches, Ho, Wo


def _flatten_w(w_conv, cin_p, kp, cp):
    cout, cin, k, _ = w_conv.shape
    w_full = jnp.zeros((cout, cin_p, k, k), jnp.float32)
    w_full = w_full.at[:, :cin].set(w_conv)
    w_mat = jnp.transpose(w_full, (2, 3, 1, 0)).reshape(k * k * cin_p, cout)
    w_mat = jnp.pad(w_mat, ((0, kp - k * k * cin_p), (0, cp - cout)))
    return w_mat.astype(jnp.bfloat16)


def _conv(x, w, b, *, stride=1, pad=0, mode='zeros', do_norm=True,
          act='relu', res=None):
    xp = _pad_hw(x, pad, mode)
    k = w.shape[2]
    patches, Ho, Wo = _im2col(xp.astype(jnp.bfloat16), k, stride)
    M = Ho * Wo
    K_raw = patches.shape[1]
    Kp = _ceil_to(K_raw, 256)
    Cp = _ceil_to(w.shape[0], 128)
    if Kp > K_raw:
        patches = jnp.pad(patches, ((0, 0), (0, Kp - K_raw)))
    w_mat = _flatten_w(w, x.shape[2], Kp, Cp)
    b_pad = jnp.pad(b, (0, Cp - b.shape[0])).astype(jnp.float32)
    res_flat = None if res is None else res.reshape(M, Cp)
    out = _gemm(patches, w_mat, b_pad, do_norm=do_norm, act=act,
                res=res_flat, m_true=M)
    return out.reshape(Ho, Wo, Cp)


def _deconv(x, w_t, b):
    H, W, C = x.shape
    xd = jnp.zeros((2 * H + 2, 2 * W + 2, C), x.dtype)
    xd = xd.at[1:2 * H:2, 1:2 * W:2, :].set(x)
    w_conv = jnp.transpose(jnp.flip(w_t, axis=(2, 3)), (1, 0, 2, 3))
    return _conv(xd, w_conv, b, stride=1, pad=0, do_norm=True, act='relu')


def kernel(x, c0_w, c0_b, d0_w, d0_b, d1_w, d1_b,
           res0_w1, res0_b1, res0_w2, res0_b2,
           res1_w1, res1_b1, res1_w2, res1_b2,
           res2_w1, res2_b1, res2_w2, res2_b2,
           res3_w1, res3_b1, res3_w2, res3_b2,
           res4_w1, res4_b1, res4_w2, res4_b2,
           res5_w1, res5_b1, res5_w2, res5_b2,
           res6_w1, res6_b1, res6_w2, res6_b2,
           res7_w1, res7_b1, res7_w2, res7_b2,
           res8_w1, res8_b1, res8_w2, res8_b2,
           u0_w, u0_b, u1_w, u1_b, cf_w, cf_b):
    res_p = [(res0_w1, res0_b1, res0_w2, res0_b2),
             (res1_w1, res1_b1, res1_w2, res1_b2),
             (res2_w1, res2_b1, res2_w2, res2_b2),
             (res3_w1, res3_b1, res3_w2, res3_b2),
             (res4_w1, res4_b1, res4_w2, res4_b2),
             (res5_w1, res5_b1, res5_w2, res5_b2),
             (res6_w1, res6_b1, res6_w2, res6_b2),
             (res7_w1, res7_b1, res7_w2, res7_b2),
             (res8_w1, res8_b1, res8_w2, res8_b2)]

    h = jnp.transpose(x[0], (1, 2, 0)).astype(jnp.float32)
    h = _conv(h, c0_w, c0_b, pad=3, mode='reflect')
    h = _conv(h, d0_w, d0_b, stride=2, pad=1)
    h = _conv(h, d1_w, d1_b, stride=2, pad=1)

    for w1, b1, w2, b2 in res_p:
        skip = h
        t = _conv(h, w1, b1, pad=1, mode='reflect')
        h = _conv(t, w2, b2, pad=1, mode='reflect', act='none', res=skip)

    h = _deconv(h, u0_w, u0_b)
    h = _deconv(h, u1_w, u1_b)
    h = _conv(h, cf_w, cf_b, pad=3, mode='reflect',
              do_norm=False, act='tanh')

    out = h[:, :, :3]
    return jnp.transpose(out, (2, 0, 1))[None]


# speedup vs baseline: 1.0910x; 1.0910x over previous
import functools

import jax
import jax.numpy as jnp
from jax import lax
from jax.experimental import pallas as pl
from jax.experimental.pallas import tpu as pltpu

_EPS = 1e-5
_VMEM = 50 * 1024 * 1024


def _ceil_to(x, m):
    return (x + m - 1) // m * m


def _gemm_stats_body(p_ref, w_ref, b_ref, y_ref, s1_ref, s2_ref):
    m = pl.program_id(0)
    val = jnp.dot(p_ref[...], w_ref[...],
                  preferred_element_type=jnp.float32) + b_ref[...]

    @pl.when(m == 0)
    def _():
        s1_ref[...] = jnp.zeros_like(s1_ref)
        s2_ref[...] = jnp.zeros_like(s2_ref)

    s1_ref[0] = s1_ref[0] + jnp.sum(val, axis=0, keepdims=True)
    s2_ref[0] = s2_ref[0] + jnp.sum(val * val, axis=0, keepdims=True)
    y_ref[...] = val


def _gemm_act_body(act, p_ref, w_ref, b_ref, y_ref):
    val = jnp.dot(p_ref[...], w_ref[...],
                  preferred_element_type=jnp.float32) + b_ref[...]
    if act == 'relu':
        val = jnp.maximum(val, 0.0)
    elif act == 'tanh':
        val = jnp.tanh(val)
    y_ref[...] = val


def _norm_body(act, add_res, inv_m, y_ref, s1_ref, s2_ref, *rest):
    if add_res:
        r_ref, o_ref = rest
    else:
        (o_ref,) = rest
    mean = s1_ref[0, 0] * inv_m
    ex2 = s2_ref[0, 0] * inv_m
    var = jnp.maximum(ex2 - mean * mean, 0.0)
    val = (y_ref[...] - mean) * lax.rsqrt(var + _EPS)
    if add_res:
        val = val + r_ref[...]
    if act == 'relu':
        val = jnp.maximum(val, 0.0)
    elif act == 'tanh':
        val = jnp.tanh(val)
    o_ref[...] = val


def _pick_tm(m):
    for cand in (2048, 1024, 512):
        if m % cand == 0:
            return cand
    return m


def _gemm(patches, w_mat, bias, *, do_norm, act, res, m_true):
    M, Kp = patches.shape
    Cp = w_mat.shape[1]
    tm = _pick_tm(M)
    n_mt = M // tm
    b2 = bias.reshape(1, Cp)

    p_spec = pl.BlockSpec((tm, Kp), lambda m: (m, 0))
    w_spec = pl.BlockSpec((Kp, Cp), lambda m: (0, 0))
    b_spec = pl.BlockSpec((1, Cp), lambda m: (0, 0))
    y_spec = pl.BlockSpec((tm, Cp), lambda m: (m, 0))
    params = pltpu.CompilerParams(
        dimension_semantics=("arbitrary",),
        vmem_limit_bytes=_VMEM)

    if not do_norm:
        return pl.pallas_call(
            functools.partial(_gemm_act_body, act),
            grid=(n_mt,),
            in_specs=[p_spec, w_spec, b_spec],
            out_specs=y_spec,
            out_shape=jax.ShapeDtypeStruct((M, Cp), jnp.float32),
            compiler_params=params,
        )(patches, w_mat, b2)

    stat_spec = pl.BlockSpec((1, 1, Cp), lambda m: (0, 0, 0))
    y, s1, s2 = pl.pallas_call(
        _gemm_stats_body,
        grid=(n_mt,),
        in_specs=[p_spec, w_spec, b_spec],
        out_specs=(y_spec, stat_spec, stat_spec),
        out_shape=(
            jax.ShapeDtypeStruct((M, Cp), jnp.float32),
            jax.ShapeDtypeStruct((1, 1, Cp), jnp.float32),
            jax.ShapeDtypeStruct((1, 1, Cp), jnp.float32),
        ),
        compiler_params=params,
    )(patches, w_mat, b2)

    norm_specs = [
        y_spec,
        pl.BlockSpec((1, 1, Cp), lambda m: (0, 0, 0)),
        pl.BlockSpec((1, 1, Cp), lambda m: (0, 0, 0)),
    ]
    args = [y, s1, s2]
    if res is not None:
        norm_specs.append(y_spec)
        args.append(res)
    return pl.pallas_call(
        functools.partial(_norm_body, act, res is not None, 1.0 / m_true),
        grid=(n_mt,),
        in_specs=norm_specs,
        out_specs=y_spec,
        out_shape=jax.ShapeDtypeStruct((M, Cp), jnp.float32),
        compiler_params=pltpu.CompilerParams(
            dimension_semantics=("arbitrary",),
            vmem_limit_bytes=_VMEM),
    )(*args)


def _pad_hw(x, p, mode):
    if p == 0:
        return x
    w = ((p, p), (p, p), (0, 0))
    return jnp.pad(x, w, mode='reflect' if mode == 'reflect' else 'constant')


def _im2col(xp, k, stride):
    H, W, C = xp.shape
    Ho = (H - k) // stride + 1
    Wo = (W - k) // stride + 1
    cols = [xp[i:i + Ho * stride:stride, j:j + Wo * stride:stride, :]
            for i in range(k) for j in range(k)]
    patches = jnp.stack(cols, axis=2).reshape(Ho * Wo, k * k * C)
    return patches, Ho, Wo


def _flatten_w(w_conv, cin_p, kp, cp):
    cout, cin, k, _ = w_conv.shape
    w_full = jnp.zeros((cout, cin_p, k, k), jnp.float32)
    w_full = w_full.at[:, :cin].set(w_conv)
    w_mat = jnp.transpose(w_full, (2, 3, 1, 0)).reshape(k * k * cin_p, cout)
    w_mat = jnp.pad(w_mat, ((0, kp - k * k * cin_p), (0, cp - cout)))
    return w_mat.astype(jnp.bfloat16)


def _conv(x, w, b, *, stride=1, pad=0, mode='zeros', do_norm=True,
          act='relu', res=None):
    cin_t = w.shape[1]
    if x.shape[2] > cin_t:
        x = x[:, :, :cin_t]
    xp = _pad_hw(x, pad, mode)
    k = w.shape[2]
    patches, Ho, Wo = _im2col(xp.astype(jnp.bfloat16), k, stride)
    M = Ho * Wo
    K_raw = patches.shape[1]
    Kp = _ceil_to(K_raw, 128)
    Cp = _ceil_to(w.shape[0], 128)
    if Kp > K_raw:
        patches = jnp.pad(patches, ((0, 0), (0, Kp - K_raw)))
    w_mat = _flatten_w(w, x.shape[2], Kp, Cp)
    b_pad = jnp.pad(b, (0, Cp - b.shape[0])).astype(jnp.float32)
    res_flat = None if res is None else res.reshape(M, Cp)
    out = _gemm(patches, w_mat, b_pad, do_norm=do_norm, act=act,
                res=res_flat, m_true=M)
    return out.reshape(Ho, Wo, Cp)


def _deconv(x, w_t, b):
    H, W, C = x.shape
    xd = jnp.zeros((2 * H + 2, 2 * W + 2, C), x.dtype)
    xd = xd.at[1:2 * H:2, 1:2 * W:2, :].set(x)
    w_conv = jnp.transpose(jnp.flip(w_t, axis=(2, 3)), (1, 0, 2, 3))
    return _conv(xd, w_conv, b, stride=1, pad=0, do_norm=True, act='relu')


def kernel(x, c0_w, c0_b, d0_w, d0_b, d1_w, d1_b,
           res0_w1, res0_b1, res0_w2, res0_b2,
           res1_w1, res1_b1, res1_w2, res1_b2,
           res2_w1, res2_b1, res2_w2, res2_b2,
           res3_w1, res3_b1, res3_w2, res3_b2,
           res4_w1, res4_b1, res4_w2, res4_b2,
           res5_w1, res5_b1, res5_w2, res5_b2,
           res6_w1, res6_b1, res6_w2, res6_b2,
           res7_w1, res7_b1, res7_w2, res7_b2,
           res8_w1, res8_b1, res8_w2, res8_b2,
           u0_w, u0_b, u1_w, u1_b, cf_w, cf_b):
    res_p = [(res0_w1, res0_b1, res0_w2, res0_b2),
             (res1_w1, res1_b1, res1_w2, res1_b2),
             (res2_w1, res2_b1, res2_w2, res2_b2),
             (res3_w1, res3_b1, res3_w2, res3_b2),
             (res4_w1, res4_b1, res4_w2, res4_b2),
             (res5_w1, res5_b1, res5_w2, res5_b2),
             (res6_w1, res6_b1, res6_w2, res6_b2),
             (res7_w1, res7_b1, res7_w2, res7_b2),
             (res8_w1, res8_b1, res8_w2, res8_b2)]

    h = jnp.transpose(x[0], (1, 2, 0)).astype(jnp.float32)
    h = _conv(h, c0_w, c0_b, pad=3, mode='reflect')
    h = _conv(h, d0_w, d0_b, stride=2, pad=1)
    h = _conv(h, d1_w, d1_b, stride=2, pad=1)

    for w1, b1, w2, b2 in res_p:
        skip = h
        t = _conv(h, w1, b1, pad=1, mode='reflect')
        h = _conv(t, w2, b2, pad=1, mode='reflect', act='none', res=skip)

    h = _deconv(h, u0_w, u0_b)
    h = _deconv(h, u1_w, u1_b)
    h = _conv(h, cf_w, cf_b, pad=3, mode='reflect',
              do_norm=False, act='tanh')

    out = h[:, :, :3]
    return jnp.transpose(out, (2, 0, 1))[None]
